# baseline (speedup 1.0000x reference)
import functools

import jax
import jax.numpy as jnp
from jax import lax
from jax.experimental import pallas as pl
from jax.experimental.pallas import tpu as pltpu
from jax.experimental.pallas import tpu_sc as plsc

_Q = 1024
_MEM = 100000
_D = 128
_K = 8
_BM = 2048
_NB = (_MEM + _BM - 1) // _BM
_NEG = float("-inf")


def _topk_body(q_ref, k_ref, ti_ref, tv_s, ti_s, qn_s, sim_s):
    j = pl.program_id(0)

    @pl.when(j == 0)
    def _():
        tv_s[...] = jnp.full((_Q, _K), _NEG, jnp.float32)
        ti_s[...] = jnp.zeros((_Q, _K), jnp.int32)
        q = q_ref[...]
        qn_s[...] = q / jnp.maximum(
            jnp.sqrt(jnp.sum(q * q, axis=1, keepdims=True)), 1e-12)

    kb = k_ref[...]
    kn = kb / jnp.maximum(
        jnp.sqrt(jnp.sum(kb * kb, axis=1, keepdims=True)), 1e-12)
    sim = lax.dot_general(qn_s[...], kn, (((1,), (1,)), ((), ())),
                          preferred_element_type=jnp.float32)
    col = lax.broadcasted_iota(jnp.int32, (_Q, _BM), 1) + j * _BM
    sim = jnp.where(col < _MEM, sim, _NEG)
    sim_s[...] = sim

    slot = lax.broadcasted_iota(jnp.int32, (_Q, _K), 1)
    big = jnp.int32(2**31 - 1)

    def _insert(m, im):
        tv = tv_s[...]
        ti = ti_s[...]
        ahead = jnp.logical_or(tv > m, jnp.logical_and(tv == m, ti < im))
        pos = jnp.sum(ahead.astype(jnp.int32), axis=1, keepdims=True)
        tv_sh = jnp.concatenate([tv[:, :1], tv[:, :-1]], axis=1)
        ti_sh = jnp.concatenate([ti[:, :1], ti[:, :-1]], axis=1)
        keep = slot < pos
        at = slot == pos
        tv_new = jnp.where(keep, tv, jnp.where(at, m, tv_sh))
        ti_new = jnp.where(keep, ti, jnp.where(at, im, ti_sh))
        tv_s[...] = tv_new
        ti_s[...] = ti_new
        return tv_new[:, _K - 1:]

    nsl = _BM // 128
    b1 = sim[:, 0:128]
    g1 = jnp.zeros((_Q, 128), jnp.int32)
    b2 = jnp.full((_Q, 128), _NEG, jnp.float32)
    g2 = jnp.zeros((_Q, 128), jnp.int32)
    for i in range(1, nsl):
        s_i = sim[:, i * 128:(i + 1) * 128]
        c1 = s_i > b1
        c2 = s_i > b2
        b2 = jnp.where(c1, b1, jnp.where(c2, s_i, b2))
        g2 = jnp.where(c1, g1, jnp.where(c2, jnp.int32(i), g2))
        b1 = jnp.where(c1, s_i, b1)
        g1 = jnp.where(c1, jnp.int32(i), g1)
    lane = lax.broadcasted_iota(jnp.int32, (_Q, 128), 1)
    jbase = j * _BM + lane
    a1 = jbase + (g1 << 7)
    a2 = jbase + (g2 << 7)
    piota = lane

    tv0 = tv_s[...]
    ti0 = ti_s[...]

    def cond(carry):
        k, m, t8, d = carry
        return jnp.logical_and(k < _K, jnp.any(m > t8))

    def body(carry):
        k, m, _, d = carry
        h = jnp.where(d == 0, b1, jnp.where(d == 1, b2, _NEG))
        ha = jnp.where(d == 0, a1, a2)
        im = jnp.min(jnp.where(h == m, ha, big), axis=1, keepdims=True)
        p = jnp.bitwise_and(im, jnp.int32(127))
        d_new = d + jnp.where(piota == p, 1, 0)
        t8 = _insert(m, im)
        h2 = jnp.where(d_new == 0, b1, jnp.where(d_new == 1, b2, _NEG))
        m_new = jnp.max(h2, axis=1, keepdims=True)
        return k + 1, m_new, t8, d_new

    m0 = jnp.max(b1, axis=1, keepdims=True)
    d0 = jnp.zeros((_Q, 128), jnp.int32)
    _, _, t8f, df = lax.while_loop(
        cond, body, (jnp.int32(0), m0, tv_s[:, _K - 1:], d0))

    bad = jnp.any(jnp.logical_and(df >= 2, b2 > t8f))

    @pl.when(bad)
    def _():
        tv_s[...] = tv0
        ti_s[...] = ti0

        def cond2(carry):
            k, m, t8 = carry
            return jnp.logical_and(k < _K, jnp.any(m > t8))

        def body2(carry):
            k, m, _ = carry
            s = sim_s[...]
            im = jnp.min(jnp.where(s == m, col, big), axis=1,
                         keepdims=True)
            masked = jnp.where(col == im, _NEG, s)
            sim_s[...] = masked
            t8 = _insert(m, im)
            m_new = jnp.max(masked, axis=1, keepdims=True)
            return k + 1, m_new, t8

        m00 = jnp.max(sim_s[...], axis=1, keepdims=True)
        lax.while_loop(cond2, body2, (jnp.int32(0), m00, tv0[:, _K - 1:]))

    @pl.when(j == _NB - 1)
    def _():
        ti_ref[...] = ti_s[...]


def _topk_indices(query_embeddings, keys):
    return pl.pallas_call(
        _topk_body,
        grid=(_NB,),
        in_specs=[
            pl.BlockSpec((_Q, _D), lambda j: (0, 0)),
            pl.BlockSpec((_BM, _D), lambda j: (j, 0)),
        ],
        out_specs=pl.BlockSpec((_Q, _K), lambda j: (0, 0)),
        out_shape=jax.ShapeDtypeStruct((_Q, _K), jnp.int32),
        scratch_shapes=[
            pltpu.VMEM((_Q, _K), jnp.float32),
            pltpu.VMEM((_Q, _K), jnp.int32),
            pltpu.VMEM((_Q, _D), jnp.float32),
            pltpu.VMEM((_Q, _BM), jnp.float32),
        ],
    )(query_embeddings, keys)


def _gather_rows(values, idx_flat):
    info = plsc.get_sparse_core_info()
    nw = info.num_cores * info.num_subcores
    b = idx_flat.shape[0]
    bpw = b // nw
    mesh = plsc.VectorSubcoreMesh(core_axis_name="c", subcore_axis_name="s")

    @functools.partial(
        pl.kernel,
        out_type=jax.ShapeDtypeStruct((b, _D), jnp.float32),
        mesh=mesh,
        scratch_types=[
            pltpu.VMEM((bpw,), jnp.int32),
            pltpu.VMEM((bpw, _D), jnp.float32),
            pltpu.SemaphoreType.DMA,
        ],
    )
    def gather(values_hbm, idx_hbm, out_hbm, idx_v, rows_v, sem):
        wid = lax.axis_index("s") * info.num_cores + lax.axis_index("c")
        base = wid * bpw
        pltpu.sync_copy(idx_hbm.at[pl.ds(base, bpw)], idx_v)
        for c in range(bpw // 128):
            pltpu.async_copy(
                values_hbm.at[idx_v.at[pl.ds(c * 128, 128)]],
                rows_v.at[pl.ds(c * 128, 128)],
                sem,
            ).wait()
        pltpu.sync_copy(rows_v, out_hbm.at[pl.ds(base, bpw)])

    return gather(values, idx_flat)


def kernel(query_embeddings, keys, values, top_k):
    del top_k
    ti = _topk_indices(query_embeddings, keys)
    rows = _gather_rows(values, ti.reshape(-1))
    return rows.reshape(_Q, _K, _D)


# speedup vs baseline: 1.1051x; 1.1051x over previous
import functools

import jax
import jax.numpy as jnp
from jax import lax
from jax.experimental import pallas as pl
from jax.experimental.pallas import tpu as pltpu
from jax.experimental.pallas import tpu_sc as plsc

_Q = 1024
_MEM = 100000
_D = 128
_K = 8
_BM = 2048
_NB = (_MEM + _BM - 1) // _BM
_NEG = float("-inf")


def _topk_body(q_ref, k_ref, ti_ref, tv_s, ti_s, qn_s, sim_s):
    j = pl.program_id(0)

    @pl.when(j == 0)
    def _():
        tv_s[...] = jnp.full((_Q, _K), _NEG, jnp.float32)
        ti_s[...] = jnp.zeros((_Q, _K), jnp.int32)
        q = q_ref[...]
        qn_s[...] = q / jnp.maximum(
            jnp.sqrt(jnp.sum(q * q, axis=1, keepdims=True)), 1e-12)

    kb = k_ref[...]
    kn = kb / jnp.maximum(
        jnp.sqrt(jnp.sum(kb * kb, axis=1, keepdims=True)), 1e-12)
    sim = lax.dot_general(qn_s[...], kn, (((1,), (1,)), ((), ())),
                          preferred_element_type=jnp.float32)
    col = lax.broadcasted_iota(jnp.int32, (_Q, _BM), 1) + j * _BM

    @pl.when(j == _NB - 1)
    def _():
        sim_s[...] = jnp.where(col < _MEM, sim, _NEG)

    @pl.when(j < _NB - 1)
    def _():
        sim_s[...] = sim

    slot = lax.broadcasted_iota(jnp.int32, (_Q, _K), 1)
    big = jnp.int32(2**31 - 1)

    def cond(carry):
        k, m, t8 = carry
        return jnp.logical_and(k < _K, jnp.any(m > t8))

    def body(carry):
        k, m, _ = carry
        s = sim_s[...]
        im = jnp.min(jnp.where(s == m, col, big), axis=1, keepdims=True)
        masked = jnp.where(col == im, _NEG, s)
        sim_s[...] = masked
        tv = tv_s[...]
        ti = ti_s[...]
        pos = jnp.sum((tv >= m).astype(jnp.int32), axis=1, keepdims=True)
        tv_sh = jnp.concatenate([tv[:, :1], tv[:, :-1]], axis=1)
        ti_sh = jnp.concatenate([ti[:, :1], ti[:, :-1]], axis=1)
        keep = slot < pos
        at = slot == pos
        tv_new = jnp.where(keep, tv, jnp.where(at, m, tv_sh))
        ti_new = jnp.where(keep, ti, jnp.where(at, im, ti_sh))
        tv_s[...] = tv_new
        ti_s[...] = ti_new
        m_new = jnp.max(masked, axis=1, keepdims=True)
        return k + 1, m_new, tv_new[:, _K - 1:]

    m0 = jnp.max(sim_s[...], axis=1, keepdims=True)
    lax.while_loop(cond, body, (jnp.int32(0), m0, tv_s[:, _K - 1:]))

    @pl.when(j == _NB - 1)
    def _():
        ti_ref[...] = ti_s[...]


def _topk_indices(query_embeddings, keys):
    return pl.pallas_call(
        _topk_body,
        grid=(_NB,),
        in_specs=[
            pl.BlockSpec((_Q, _D), lambda j: (0, 0)),
            pl.BlockSpec((_BM, _D), lambda j: (j, 0)),
        ],
        out_specs=pl.BlockSpec((_Q, _K), lambda j: (0, 0)),
        out_shape=jax.ShapeDtypeStruct((_Q, _K), jnp.int32),
        scratch_shapes=[
            pltpu.VMEM((_Q, _K), jnp.float32),
            pltpu.VMEM((_Q, _K), jnp.int32),
            pltpu.VMEM((_Q, _D), jnp.float32),
            pltpu.VMEM((_Q, _BM), jnp.float32),
        ],
    )(query_embeddings, keys)


def _gather_rows(values, idx_flat):
    info = plsc.get_sparse_core_info()
    nw = info.num_cores * info.num_subcores
    b = idx_flat.shape[0]
    bpw = b // nw
    mesh = plsc.VectorSubcoreMesh(core_axis_name="c", subcore_axis_name="s")

    @functools.partial(
        pl.kernel,
        out_type=jax.ShapeDtypeStruct((b, _D), jnp.float32),
        mesh=mesh,
        scratch_types=[
            pltpu.VMEM((bpw,), jnp.int32),
            pltpu.VMEM((bpw, _D), jnp.float32),
            pltpu.SemaphoreType.DMA,
        ],
    )
    def gather(values_hbm, idx_hbm, out_hbm, idx_v, rows_v, sem):
        wid = lax.axis_index("s") * info.num_cores + lax.axis_index("c")
        base = wid * bpw
        pltpu.sync_copy(idx_hbm.at[pl.ds(base, bpw)], idx_v)
        for c in range(bpw // 128):
            pltpu.async_copy(
                values_hbm.at[idx_v.at[pl.ds(c * 128, 128)]],
                rows_v.at[pl.ds(c * 128, 128)],
                sem,
            ).wait()
        pltpu.sync_copy(rows_v, out_hbm.at[pl.ds(base, bpw)])

    return gather(values, idx_flat)


def kernel(query_embeddings, keys, values, top_k):
    del top_k
    ti = _topk_indices(query_embeddings, keys)
    rows = _gather_rows(values, ti.reshape(-1))
    return rows.reshape(_Q, _K, _D)
